# baseline (speedup 1.0000x reference)
import functools

import jax
import jax.numpy as jnp
from jax import lax
from jax.experimental import pallas as pl
from jax.experimental.pallas import tpu as pltpu
from jax.experimental.pallas import tpu_sc as plsc

F32 = jnp.float32
N = 2048
H = 64
K = 30
NRBF = 16

RB1 = 256
RBM = 256
RBU = 512

NC, NS = 2, 16
NW = NC * NS
CH = 128

_CSP = 20.0 / 15.0
_INV2W2 = 1.0 / (2.0 * (20.0 / 16.0) ** 2)


def _centers(rows):
    c = lax.broadcasted_iota(jnp.int32, (rows, NRBF), 1)
    return c.astype(F32) * _CSP


def _silu(z):
    return z / (1.0 + jnp.exp(-z))


def _s1_body(c_ref, ct_ref, idx_ref, de_ref, rbf_ref):
    b = pl.program_id(0)
    i = pl.program_id(1)
    cr = c_ref[0]
    ct = ct_ref[0]
    sqr = jnp.sum(cr * cr, axis=1, keepdims=True)
    sqc = jnp.sum(ct * ct, axis=0, keepdims=True)
    dot = jnp.dot(cr, ct, preferred_element_type=F32)
    d2 = sqr + sqc - 2.0 * dot
    dist = jnp.sqrt(jnp.maximum(d2, 1e-12))
    col = lax.broadcasted_iota(jnp.int32, (RB1, N), 1)
    rowg = i * RB1 + lax.broadcasted_iota(jnp.int32, (RB1, N), 0)
    D = jnp.where(col == rowg, dist + 1e10, dist)
    mean = jnp.sum(dist, axis=1, keepdims=True) * (1.0 / N)
    rbf_ref[0] = jnp.exp(-(mean - _centers(RB1)) ** 2 * _INV2W2)
    idx_cols, d_cols = [], []
    for _ in range(K):
        mval = jnp.min(D, axis=1, keepdims=True)
        hit = D == mval
        jstar = jnp.min(jnp.where(hit, col, N), axis=1, keepdims=True)
        idx_cols.append(jstar)
        d_cols.append(mval)
        D = jnp.where(col == jstar, 1e30, D)
    idx_ref[0] = jnp.concatenate(idx_cols, axis=1) + b * N
    de_ref[0] = jnp.concatenate(d_cols, axis=1)


def _s1_call(coords, coordsT):
    B = coords.shape[0]
    return pl.pallas_call(
        _s1_body,
        grid=(B, N // RB1),
        in_specs=[
            pl.BlockSpec((1, RB1, 3), lambda b, i: (b, i, 0)),
            pl.BlockSpec((1, 3, N), lambda b, i: (b, 0, 0)),
        ],
        out_specs=[
            pl.BlockSpec((1, RB1, K), lambda b, i: (b, i, 0)),
            pl.BlockSpec((1, RB1, K), lambda b, i: (b, i, 0)),
            pl.BlockSpec((1, RB1, NRBF), lambda b, i: (b, i, 0)),
        ],
        out_shape=[
            jax.ShapeDtypeStruct((B, N, K), jnp.int32),
            jax.ShapeDtypeStruct((B, N, K), F32),
            jax.ShapeDtypeStruct((B, N, NRBF), F32),
        ],
    )(coords, coordsT)


def _s2_body(vp_ref, vs_ref, rbf_ref, aw_ref, ab_ref, pwa_ref, pwr_ref,
             pb_ref, w1a_ref, b1_ref, w1b_ref, h_ref, a_ref, bm_ref):
    vp = vp_ref[0]
    vs = vs_ref[0]
    npn = jnp.sqrt(jnp.sum(vp * vp, axis=1, keepdims=True))
    nsn = jnp.sqrt(jnp.sum(vs * vs, axis=1, keepdims=True))
    vpn = vp / jnp.maximum(npn, 1e-12)
    vsn = vs / jnp.maximum(nsn, 1e-12)
    cos = jnp.clip(jnp.sum(vpn * vsn, axis=1, keepdims=True), -1.0, 1.0)
    sin = jnp.sqrt(jnp.maximum(1.0 - cos * cos, 0.0))
    ang = jnp.concatenate([sin, cos, cos, sin], axis=1)
    af = jnp.dot(ang, aw_ref[...], preferred_element_type=F32) + ab_ref[...]
    h = (jnp.dot(af, pwa_ref[...], preferred_element_type=F32)
         + jnp.dot(rbf_ref[0], pwr_ref[...], preferred_element_type=F32)
         + pb_ref[...])
    h_ref[0] = h
    a_ref[0] = jnp.dot(h, w1a_ref[...], preferred_element_type=F32) + b1_ref[...]
    bm_ref[0] = jnp.dot(h, w1b_ref[...], preferred_element_type=F32).astype(jnp.bfloat16)


def _s2_call(v1p, v1s, rbfm, aw4, ab, pwa, pwr, pb, w1a, b1, w1b):
    B = v1p.shape[0]
    full = lambda b, i: (0, 0)
    return pl.pallas_call(
        _s2_body,
        grid=(B, N // RBU),
        in_specs=[
            pl.BlockSpec((1, RBU, 3), lambda b, i: (b, i, 0)),
            pl.BlockSpec((1, RBU, 3), lambda b, i: (b, i, 0)),
            pl.BlockSpec((1, RBU, NRBF), lambda b, i: (b, i, 0)),
            pl.BlockSpec((4, H), full),
            pl.BlockSpec((1, H), full),
            pl.BlockSpec((H, H), full),
            pl.BlockSpec((NRBF, H), full),
            pl.BlockSpec((1, H), full),
            pl.BlockSpec((H, H), full),
            pl.BlockSpec((1, H), full),
            pl.BlockSpec((H, H), full),
        ],
        out_specs=[pl.BlockSpec((1, RBU, H), lambda b, i: (b, i, 0))] * 3,
        out_shape=[jax.ShapeDtypeStruct((B, N, H), F32)] * 2
        + [jax.ShapeDtypeStruct((B, N, H), jnp.bfloat16)],
    )(v1p, v1s, rbfm, aw4, ab, pwa, pwr, pb, w1a, b1, w1b)


def _msg_body(a_ref, bd_ref, de_ref, w1c_ref, w2_ref, b2_ref, m_ref):
    a = a_ref[...]
    de = de_ref[...]
    w1c = w1c_ref[...]
    w2 = w2_ref[...]
    b2 = b2_ref[...]
    cen = _centers(RBM)
    for k in range(K):
        dk = de[:, k:k + 1]
        e16 = jnp.exp(-(dk - cen) ** 2 * _INV2W2)
        z = (a + bd_ref[k].astype(F32)
             + jnp.dot(e16, w1c, preferred_element_type=F32))
        m_ref[k] = jnp.dot(_silu(z), w2, preferred_element_type=F32) + b2


def _msg_call(a, bd, de, w1c, w2, b2):
    BN = a.shape[0]
    full = lambda i: (0, 0)
    return pl.pallas_call(
        _msg_body,
        grid=(BN // RBM,),
        in_specs=[
            pl.BlockSpec((RBM, H), lambda i: (i, 0)),
            pl.BlockSpec((K, RBM, H), lambda i: (0, i, 0)),
            pl.BlockSpec((RBM, K), lambda i: (i, 0)),
            pl.BlockSpec((NRBF, H), full),
            pl.BlockSpec((H, H), full),
            pl.BlockSpec((1, H), full),
        ],
        out_specs=pl.BlockSpec((K, RBM, H), lambda i: (0, i, 0)),
        out_shape=jax.ShapeDtypeStruct((K, BN, H), F32),
    )(a, bd, de, w1c, w2, b2)


def _ln_rows(x, g, b):
    mu = jnp.mean(x, axis=1, keepdims=True)
    xc = x - mu
    var = jnp.mean(xc * xc, axis=1, keepdims=True)
    return xc / jnp.sqrt(var + 1e-5) * g + b


def _upd_common(h_ref, p_ref, u1a_ref, u1b_ref, b1u_ref, u2_ref, b2u_ref,
                g_ref, bb_ref):
    h = h_ref[...]
    aggr = p_ref[0] + p_ref[1]
    z = (jnp.dot(h, u1a_ref[...], preferred_element_type=F32)
         + jnp.dot(aggr, u1b_ref[...], preferred_element_type=F32)
         + b1u_ref[...])
    u = jnp.dot(_silu(z), u2_ref[...], preferred_element_type=F32) + b2u_ref[...]
    return _ln_rows(h + u, g_ref[...], bb_ref[...])


def _upd_mid_body(h_ref, p_ref, u1a_ref, u1b_ref, b1u_ref, u2_ref, b2u_ref,
                  g_ref, bb_ref, w1a_ref, b1n_ref, w1b_ref,
                  hn_ref, an_ref, bmn_ref):
    hn = _upd_common(h_ref, p_ref, u1a_ref, u1b_ref, b1u_ref, u2_ref,
                     b2u_ref, g_ref, bb_ref)
    hn_ref[...] = hn
    an_ref[...] = jnp.dot(hn, w1a_ref[...], preferred_element_type=F32) + b1n_ref[...]
    bmn_ref[...] = jnp.dot(hn, w1b_ref[...], preferred_element_type=F32).astype(jnp.bfloat16)


def _upd_last_body(h_ref, p_ref, u1a_ref, u1b_ref, b1u_ref, u2_ref, b2u_ref,
                   g_ref, bb_ref, og_ref, ob_ref, ow_ref, ob2_ref, out_ref):
    hn = _upd_common(h_ref, p_ref, u1a_ref, u1b_ref, b1u_ref, u2_ref,
                     b2u_ref, g_ref, bb_ref)
    y = _ln_rows(hn, og_ref[...], ob_ref[...])
    out_ref[...] = jnp.dot(y, ow_ref[...], preferred_element_type=F32) + ob2_ref[...]


def _upd_call(h, p, consts, last):
    BN = h.shape[0]
    full = lambda i: (0, 0)
    nconst = len(consts)
    in_specs = [
        pl.BlockSpec((RBU, H), lambda i: (i, 0)),
        pl.BlockSpec((2, RBU, H), lambda i: (0, i, 0)),
    ] + [pl.BlockSpec(c.shape, full) for c in consts]
    if last:
        out_specs = pl.BlockSpec((RBU, H), lambda i: (i, 0))
        out_shape = jax.ShapeDtypeStruct((BN, H), F32)
        body = _upd_last_body
    else:
        out_specs = [pl.BlockSpec((RBU, H), lambda i: (i, 0))] * 3
        out_shape = [jax.ShapeDtypeStruct((BN, H), F32)] * 2 + [
            jax.ShapeDtypeStruct((BN, H), jnp.bfloat16)]
        body = _upd_mid_body
    return pl.pallas_call(
        body,
        grid=(BN // RBU,),
        in_specs=in_specs,
        out_specs=out_specs,
        out_shape=out_shape,
    )(h, p, *consts)


def _sc_gather(table, idx):
    E = idx.shape[0]
    per = E // NW
    steps = per // CH
    dt = table.dtype
    mesh = plsc.VectorSubcoreMesh(core_axis_name="c", subcore_axis_name="s")

    @functools.partial(
        pl.kernel,
        out_type=jax.ShapeDtypeStruct((E, H), dt),
        mesh=mesh,
        compiler_params=pltpu.CompilerParams(use_tc_tiling_on_sc=False),
        scratch_types=[
            pltpu.VMEM((CH,), jnp.int32),
            pltpu.VMEM((CH, H), dt),
            pltpu.SemaphoreType.DMA,
        ],
    )
    def gk(tab_hbm, idx_hbm, out_hbm, idx_v, rows_v, sem):
        wid = lax.axis_index("s") * NC + lax.axis_index("c")
        base = wid * per

        def step(t, carry):
            off = base + t * CH
            pltpu.sync_copy(idx_hbm.at[pl.ds(off, CH)], idx_v)
            pltpu.async_copy(tab_hbm.at[idx_v], rows_v, sem).wait()
            pltpu.sync_copy(rows_v, out_hbm.at[pl.ds(off, CH)])
            return carry

        lax.fori_loop(0, steps, step, 0)

    return gk(table, idx)


def _sc_scatter(m, idx, zeros):
    E = idx.shape[0]
    per = E // NW
    steps = per // CH
    BN = zeros.shape[0]
    rps = BN // NS
    mesh = plsc.VectorSubcoreMesh(core_axis_name="c", subcore_axis_name="s")

    @functools.partial(
        pl.kernel,
        out_type=jax.ShapeDtypeStruct((NC * BN, H), F32),
        mesh=mesh,
        compiler_params=pltpu.CompilerParams(use_tc_tiling_on_sc=False),
        scratch_types=[
            pltpu.VMEM_SHARED((BN, H), F32),
            pltpu.VMEM((CH,), jnp.int32),
            pltpu.VMEM((CH, H), F32),
        ],
    )
    def sk(m_hbm, idx_hbm, z_hbm, out_hbm, shared, idx_v, m_v):
        c = lax.axis_index("c")
        s = lax.axis_index("s")
        wid = s * NC + c
        base = wid * per

        @pl.when(s == 0)
        def _():
            pltpu.sync_copy(z_hbm, shared)

        plsc.subcore_barrier()

        def step(t, carry):
            off = base + t * CH
            pltpu.sync_copy(idx_hbm.at[pl.ds(off, CH)], idx_v)
            pltpu.sync_copy(m_hbm.at[pl.ds(off, CH)], m_v)
            pltpu.sync_copy(m_v, shared.at[idx_v], add=True)
            return carry

        lax.fori_loop(0, steps, step, 0)
        plsc.subcore_barrier()
        pltpu.sync_copy(shared.at[pl.ds(s * rps, rps)],
                        out_hbm.at[pl.ds(c * BN + s * rps, rps)])

    return sk(m, idx, zeros)


def kernel(coords, mask, angle_W, angle_b, pos_W, pos_b, msg1_W, msg1_b,
           msg2_W, msg2_b, upd1_W, upd1_b, upd2_W, upd2_b, ln_g, ln_b,
           out_ln_g, out_ln_b, out_W, out_b):
    B = coords.shape[0]
    BN = B * N
    E = BN * K

    coordsT = jnp.transpose(coords, (0, 2, 1))
    idxg, d_e, rbfm = _s1_call(coords, coordsT)

    v1 = coords[:, 1:] - coords[:, :-1]
    v1p = jnp.pad(v1, ((0, 0), (0, 1), (0, 0)))
    v1s = jnp.pad(v1, ((0, 0), (1, 0), (0, 0)))

    h, a, bm = _s2_call(v1p, v1s, rbfm, angle_W[:4], angle_b[None],
                        pos_W[:H], pos_W[H:], pos_b[None],
                        msg1_W[0, :H], msg1_b[0][None], msg1_W[0, H:2 * H])

    idx_flat = idxg.reshape(BN, K)
    dstK = idx_flat.T.reshape(-1)
    de_flat = d_e.reshape(BN, K)
    zeros = jnp.zeros((BN, H), F32)

    hf = h.reshape(BN, H)
    a = a.reshape(BN, H)
    bm = bm.reshape(BN, H)

    out = None
    for l in range(3):
        bd = _sc_gather(bm, dstK).reshape(K, BN, H)
        m = _msg_call(a, bd, de_flat, msg1_W[l, 2 * H:], msg2_W[l],
                      msg2_b[l][None])
        p = _sc_scatter(m.reshape(E, H), dstK, zeros).reshape(2, BN, H)
        if l < 2:
            hf, a, bm = _upd_call(hf, p, [
                upd1_W[l, :H], upd1_W[l, H:], upd1_b[l][None], upd2_W[l],
                upd2_b[l][None], ln_g[l][None], ln_b[l][None],
                msg1_W[l + 1, :H], msg1_b[l + 1][None],
                msg1_W[l + 1, H:2 * H]], last=False)
        else:
            out = _upd_call(hf, p, [
                upd1_W[l, :H], upd1_W[l, H:], upd1_b[l][None], upd2_W[l],
                upd2_b[l][None], ln_g[l][None], ln_b[l][None],
                out_ln_g[None], out_ln_b[None], out_W, out_b[None]],
                last=True)

    src = jnp.broadcast_to(
        jnp.arange(BN, dtype=jnp.int32)[:, None], (BN, K)).reshape(-1)
    edge_index = jnp.stack([src, idx_flat.reshape(-1)])
    return out.reshape(B, N, H), edge_index


# speedup vs baseline: 1.2389x; 1.2389x over previous
import functools

import jax
import jax.numpy as jnp
from jax import lax
from jax.experimental import pallas as pl
from jax.experimental.pallas import tpu as pltpu
from jax.experimental.pallas import tpu_sc as plsc

F32 = jnp.float32
N = 2048
H = 64
K = 30
NRBF = 16

RB1 = 256
RBM = 256
RBU = 512

NC, NS = 2, 16
NW = NC * NS
CH = 128

_CSP = 20.0 / 15.0
_INV2W2 = 1.0 / (2.0 * (20.0 / 16.0) ** 2)


def _centers(rows):
    c = lax.broadcasted_iota(jnp.int32, (rows, NRBF), 1)
    return c.astype(F32) * _CSP


def _silu(z):
    return z / (1.0 + jnp.exp(-z))


def _s1_body(c_ref, ct_ref, idx_ref, de_ref, rbf_ref):
    b = pl.program_id(0)
    i = pl.program_id(1)
    cr = c_ref[0]
    ct = ct_ref[0]
    sqr = jnp.sum(cr * cr, axis=1, keepdims=True)
    sqc = jnp.sum(ct * ct, axis=0, keepdims=True)
    dot = jnp.dot(cr, ct, preferred_element_type=F32)
    d2 = sqr + sqc - 2.0 * dot
    dist = jnp.sqrt(jnp.maximum(d2, 1e-12))
    col = lax.broadcasted_iota(jnp.int32, (RB1, N), 1)
    rowg = i * RB1 + lax.broadcasted_iota(jnp.int32, (RB1, N), 0)
    D = jnp.where(col == rowg, dist + 1e10, dist)
    mean = jnp.sum(dist, axis=1, keepdims=True) * (1.0 / N)
    rbf_ref[0] = jnp.exp(-(mean - _centers(RB1)) ** 2 * _INV2W2)
    idx_cols, d_cols = [], []
    for _ in range(K):
        mval = jnp.min(D, axis=1, keepdims=True)
        hit = D == mval
        jstar = jnp.min(jnp.where(hit, col, N), axis=1, keepdims=True)
        idx_cols.append(jstar)
        d_cols.append(mval)
        D = jnp.where(col == jstar, 1e30, D)
    idx_ref[0] = jnp.concatenate(idx_cols, axis=1) + b * N
    de_ref[0] = jnp.concatenate(d_cols, axis=1)


def _s1_call(coords, coordsT):
    B = coords.shape[0]
    return pl.pallas_call(
        _s1_body,
        grid=(B, N // RB1),
        in_specs=[
            pl.BlockSpec((1, RB1, 3), lambda b, i: (b, i, 0)),
            pl.BlockSpec((1, 3, N), lambda b, i: (b, 0, 0)),
        ],
        out_specs=[
            pl.BlockSpec((1, RB1, K), lambda b, i: (b, i, 0)),
            pl.BlockSpec((1, RB1, K), lambda b, i: (b, i, 0)),
            pl.BlockSpec((1, RB1, NRBF), lambda b, i: (b, i, 0)),
        ],
        out_shape=[
            jax.ShapeDtypeStruct((B, N, K), jnp.int32),
            jax.ShapeDtypeStruct((B, N, K), F32),
            jax.ShapeDtypeStruct((B, N, NRBF), F32),
        ],
    )(coords, coordsT)


def _s2_body(vp_ref, vs_ref, rbf_ref, aw_ref, ab_ref, pwa_ref, pwr_ref,
             pb_ref, w1a_ref, b1_ref, w1b_ref, h_ref, a_ref, bm_ref):
    vp = vp_ref[0]
    vs = vs_ref[0]
    npn = jnp.sqrt(jnp.sum(vp * vp, axis=1, keepdims=True))
    nsn = jnp.sqrt(jnp.sum(vs * vs, axis=1, keepdims=True))
    vpn = vp / jnp.maximum(npn, 1e-12)
    vsn = vs / jnp.maximum(nsn, 1e-12)
    cos = jnp.clip(jnp.sum(vpn * vsn, axis=1, keepdims=True), -1.0, 1.0)
    sin = jnp.sqrt(jnp.maximum(1.0 - cos * cos, 0.0))
    ang = jnp.concatenate([sin, cos, cos, sin], axis=1)
    af = jnp.dot(ang, aw_ref[...], preferred_element_type=F32) + ab_ref[...]
    h = (jnp.dot(af, pwa_ref[...], preferred_element_type=F32)
         + jnp.dot(rbf_ref[0], pwr_ref[...], preferred_element_type=F32)
         + pb_ref[...])
    h_ref[0] = h
    a_ref[0] = jnp.dot(h, w1a_ref[...], preferred_element_type=F32) + b1_ref[...]
    bm_ref[0] = jnp.dot(h, w1b_ref[...], preferred_element_type=F32)


def _s2_call(v1p, v1s, rbfm, aw4, ab, pwa, pwr, pb, w1a, b1, w1b):
    B = v1p.shape[0]
    full = lambda b, i: (0, 0)
    return pl.pallas_call(
        _s2_body,
        grid=(B, N // RBU),
        in_specs=[
            pl.BlockSpec((1, RBU, 3), lambda b, i: (b, i, 0)),
            pl.BlockSpec((1, RBU, 3), lambda b, i: (b, i, 0)),
            pl.BlockSpec((1, RBU, NRBF), lambda b, i: (b, i, 0)),
            pl.BlockSpec((4, H), full),
            pl.BlockSpec((1, H), full),
            pl.BlockSpec((H, H), full),
            pl.BlockSpec((NRBF, H), full),
            pl.BlockSpec((1, H), full),
            pl.BlockSpec((H, H), full),
            pl.BlockSpec((1, H), full),
            pl.BlockSpec((H, H), full),
        ],
        out_specs=[pl.BlockSpec((1, RBU, H), lambda b, i: (b, i, 0))] * 3,
        out_shape=[jax.ShapeDtypeStruct((B, N, H), F32)] * 3,
    )(v1p, v1s, rbfm, aw4, ab, pwa, pwr, pb, w1a, b1, w1b)


def _msg_body(a_ref, bd_ref, de_ref, w1c_ref, w2_ref, b2_ref, m_ref):
    a = a_ref[...]
    de = de_ref[...]
    w1c = w1c_ref[...]
    w2 = w2_ref[...]
    b2 = b2_ref[...]
    cen = _centers(RBM)
    for k in range(K):
        dk = de[:, k:k + 1]
        e16 = jnp.exp(-(dk - cen) ** 2 * _INV2W2)
        z = a + bd_ref[k] + jnp.dot(e16, w1c, preferred_element_type=F32)
        m_ref[k] = jnp.dot(_silu(z), w2, preferred_element_type=F32) + b2


def _msg_call(a, bd, de, w1c, w2, b2):
    BN = a.shape[0]
    full = lambda i: (0, 0)
    return pl.pallas_call(
        _msg_body,
        grid=(BN // RBM,),
        in_specs=[
            pl.BlockSpec((RBM, H), lambda i: (i, 0)),
            pl.BlockSpec((K, RBM, H), lambda i: (0, i, 0)),
            pl.BlockSpec((RBM, K), lambda i: (i, 0)),
            pl.BlockSpec((NRBF, H), full),
            pl.BlockSpec((H, H), full),
            pl.BlockSpec((1, H), full),
        ],
        out_specs=pl.BlockSpec((K, RBM, H), lambda i: (0, i, 0)),
        out_shape=jax.ShapeDtypeStruct((K, BN, H), F32),
    )(a, bd, de, w1c, w2, b2)


def _ln_rows(x, g, b):
    mu = jnp.mean(x, axis=1, keepdims=True)
    xc = x - mu
    var = jnp.mean(xc * xc, axis=1, keepdims=True)
    return xc / jnp.sqrt(var + 1e-5) * g + b


def _upd_common(h_ref, p_ref, u1a_ref, u1b_ref, b1u_ref, u2_ref, b2u_ref,
                g_ref, bb_ref):
    h = h_ref[...]
    aggr = p_ref[0] + p_ref[1]
    z = (jnp.dot(h, u1a_ref[...], preferred_element_type=F32)
         + jnp.dot(aggr, u1b_ref[...], preferred_element_type=F32)
         + b1u_ref[...])
    u = jnp.dot(_silu(z), u2_ref[...], preferred_element_type=F32) + b2u_ref[...]
    return _ln_rows(h + u, g_ref[...], bb_ref[...])


def _upd_mid_body(h_ref, p_ref, u1a_ref, u1b_ref, b1u_ref, u2_ref, b2u_ref,
                  g_ref, bb_ref, w1a_ref, b1n_ref, w1b_ref,
                  hn_ref, an_ref, bmn_ref):
    hn = _upd_common(h_ref, p_ref, u1a_ref, u1b_ref, b1u_ref, u2_ref,
                     b2u_ref, g_ref, bb_ref)
    hn_ref[...] = hn
    an_ref[...] = jnp.dot(hn, w1a_ref[...], preferred_element_type=F32) + b1n_ref[...]
    bmn_ref[...] = jnp.dot(hn, w1b_ref[...], preferred_element_type=F32)


def _upd_last_body(h_ref, p_ref, u1a_ref, u1b_ref, b1u_ref, u2_ref, b2u_ref,
                   g_ref, bb_ref, og_ref, ob_ref, ow_ref, ob2_ref, out_ref):
    hn = _upd_common(h_ref, p_ref, u1a_ref, u1b_ref, b1u_ref, u2_ref,
                     b2u_ref, g_ref, bb_ref)
    y = _ln_rows(hn, og_ref[...], ob_ref[...])
    out_ref[...] = jnp.dot(y, ow_ref[...], preferred_element_type=F32) + ob2_ref[...]


def _upd_call(h, p, consts, last):
    BN = h.shape[0]
    full = lambda i: (0, 0)
    nconst = len(consts)
    in_specs = [
        pl.BlockSpec((RBU, H), lambda i: (i, 0)),
        pl.BlockSpec((2, RBU, H), lambda i: (0, i, 0)),
    ] + [pl.BlockSpec(c.shape, full) for c in consts]
    if last:
        out_specs = pl.BlockSpec((RBU, H), lambda i: (i, 0))
        out_shape = jax.ShapeDtypeStruct((BN, H), F32)
        body = _upd_last_body
    else:
        out_specs = [pl.BlockSpec((RBU, H), lambda i: (i, 0))] * 3
        out_shape = [jax.ShapeDtypeStruct((BN, H), F32)] * 3
        body = _upd_mid_body
    return pl.pallas_call(
        body,
        grid=(BN // RBU,),
        in_specs=in_specs,
        out_specs=out_specs,
        out_shape=out_shape,
    )(h, p, *consts)


GRP = 6
NGRP = 10
NPAIR = NGRP // 2


def _sc_gather(table, idx3):
    NWk, steps, CHk = idx3.shape
    per = steps * CHk
    E = NWk * per
    mesh = plsc.VectorSubcoreMesh(core_axis_name="c", subcore_axis_name="s")

    @functools.partial(
        pl.kernel,
        out_type=jax.ShapeDtypeStruct((E, H), F32),
        mesh=mesh,
        compiler_params=pltpu.CompilerParams(use_tc_tiling_on_sc=False),
        scratch_types=[
            pltpu.VMEM((steps, CHk), jnp.int32),
            pltpu.VMEM((GRP * CHk, H), F32),
            pltpu.VMEM((GRP * CHk, H), F32),
            pltpu.SemaphoreType.DMA,
            pltpu.SemaphoreType.DMA,
            pltpu.SemaphoreType.DMA,
            pltpu.SemaphoreType.DMA,
        ],
    )
    def gk(tab_hbm, idx_hbm, out_hbm, idx_all, buf_a, buf_b, gs_a, gs_b,
           os_a, os_b):
        wid = lax.axis_index("s") * NC + lax.axis_index("c")
        base = wid * per
        pltpu.sync_copy(idx_hbm.at[wid], idx_all)

        def issue_group(g, buf, gs):
            for j in range(GRP):
                pltpu.async_copy(tab_hbm.at[idx_all.at[g * GRP + j]],
                                 buf.at[pl.ds(j * CHk, CHk)], gs)

        def drain_group(buf, gs):
            for j in range(GRP):
                pltpu.make_async_copy(tab_hbm.at[idx_all.at[0]],
                                      buf.at[pl.ds(j * CHk, CHk)], gs).wait()

        def out_group(g, buf, osem):
            pltpu.async_copy(
                buf, out_hbm.at[pl.ds(base + g * GRP * CHk, GRP * CHk)], osem)

        def drain_out(buf, osem):
            pltpu.make_async_copy(
                buf, out_hbm.at[pl.ds(base, GRP * CHk)], osem).wait()

        issue_group(0, buf_a, gs_a)
        issue_group(1, buf_b, gs_b)

        def body(p, carry):
            ga = 2 * p
            drain_group(buf_a, gs_a)
            out_group(ga, buf_a, os_a)
            drain_group(buf_b, gs_b)
            out_group(ga + 1, buf_b, os_b)

            @pl.when(p < NPAIR - 1)
            def _():
                drain_out(buf_a, os_a)
                issue_group(ga + 2, buf_a, gs_a)
                drain_out(buf_b, os_b)
                issue_group(ga + 3, buf_b, gs_b)

            return carry

        lax.fori_loop(0, NPAIR, body, 0)
        drain_out(buf_a, os_a)
        drain_out(buf_b, os_b)

    return gk(table, idx3)


def _sc_scatter(m, idx3, zeros):
    NWk, steps, CHk = idx3.shape
    per = steps * CHk
    rps = zeros.shape[0]
    BN = rps * NS
    mesh = plsc.VectorSubcoreMesh(core_axis_name="c", subcore_axis_name="s")

    @functools.partial(
        pl.kernel,
        out_type=jax.ShapeDtypeStruct((NC * BN, H), F32),
        mesh=mesh,
        compiler_params=pltpu.CompilerParams(use_tc_tiling_on_sc=False),
        scratch_types=[
            pltpu.VMEM_SHARED((BN, H), F32),
            pltpu.VMEM((steps, CHk), jnp.int32),
            pltpu.VMEM((CHk, H), F32),
            pltpu.VMEM((CHk, H), F32),
            pltpu.SemaphoreType.DMA,
            pltpu.SemaphoreType.DMA,
        ],
    )
    def sk(m_hbm, idx_hbm, z_hbm, out_hbm, shared, idx_all, m_a, m_b,
           ms_a, ms_b):
        c = lax.axis_index("c")
        s = lax.axis_index("s")
        wid = s * NC + c
        base = wid * per
        pltpu.sync_copy(z_hbm, shared.at[pl.ds(s * rps, rps)])
        pltpu.sync_copy(idx_hbm.at[wid], idx_all)
        plsc.subcore_barrier()

        pltpu.async_copy(m_hbm.at[pl.ds(base, CHk)], m_a, ms_a)
        pltpu.async_copy(m_hbm.at[pl.ds(base + CHk, CHk)], m_b, ms_b)

        def body(q, carry):
            t = 2 * q
            pltpu.make_async_copy(m_hbm.at[pl.ds(base, CHk)], m_a, ms_a).wait()
            pltpu.sync_copy(m_a, shared.at[idx_all.at[t]], add=True)

            @pl.when(q < steps // 2 - 1)
            def _():
                pltpu.async_copy(
                    m_hbm.at[pl.ds(base + (t + 2) * CHk, CHk)], m_a, ms_a)

            pltpu.make_async_copy(m_hbm.at[pl.ds(base, CHk)], m_b, ms_b).wait()
            pltpu.sync_copy(m_b, shared.at[idx_all.at[t + 1]], add=True)

            @pl.when(q < steps // 2 - 1)
            def _():
                pltpu.async_copy(
                    m_hbm.at[pl.ds(base + (t + 3) * CHk, CHk)], m_b, ms_b)

            return carry

        lax.fori_loop(0, steps // 2, body, 0)
        plsc.subcore_barrier()
        pltpu.sync_copy(shared.at[pl.ds(s * rps, rps)],
                        out_hbm.at[pl.ds(c * BN + s * rps, rps)])

    return sk(m, idx3, zeros)


def kernel(coords, mask, angle_W, angle_b, pos_W, pos_b, msg1_W, msg1_b,
           msg2_W, msg2_b, upd1_W, upd1_b, upd2_W, upd2_b, ln_g, ln_b,
           out_ln_g, out_ln_b, out_W, out_b):
    B = coords.shape[0]
    BN = B * N
    E = BN * K

    coordsT = jnp.transpose(coords, (0, 2, 1))
    idxg, d_e, rbfm = _s1_call(coords, coordsT)

    v1 = coords[:, 1:] - coords[:, :-1]
    v1p = jnp.pad(v1, ((0, 0), (0, 1), (0, 0)))
    v1s = jnp.pad(v1, ((0, 0), (1, 0), (0, 0)))

    h, a, bm = _s2_call(v1p, v1s, rbfm, angle_W[:4], angle_b[None],
                        pos_W[:H], pos_W[H:], pos_b[None],
                        msg1_W[0, :H], msg1_b[0][None], msg1_W[0, H:2 * H])

    idx_flat = idxg.reshape(BN, K)
    dst3 = idx_flat.T.reshape(NW, E // (NW * CH), CH)
    de_flat = d_e.reshape(BN, K)
    zeros = jnp.zeros((BN // NS, H), F32)

    hf = h.reshape(BN, H)
    a = a.reshape(BN, H)
    bm = bm.reshape(BN, H)

    out = None
    for l in range(3):
        bd = _sc_gather(bm, dst3).reshape(K, BN, H)
        m = _msg_call(a, bd, de_flat, msg1_W[l, 2 * H:], msg2_W[l],
                      msg2_b[l][None])
        p = _sc_scatter(m.reshape(E, H), dst3, zeros).reshape(2, BN, H)
        if l < 2:
            hf, a, bm = _upd_call(hf, p, [
                upd1_W[l, :H], upd1_W[l, H:], upd1_b[l][None], upd2_W[l],
                upd2_b[l][None], ln_g[l][None], ln_b[l][None],
                msg1_W[l + 1, :H], msg1_b[l + 1][None],
                msg1_W[l + 1, H:2 * H]], last=False)
        else:
            out = _upd_call(hf, p, [
                upd1_W[l, :H], upd1_W[l, H:], upd1_b[l][None], upd2_W[l],
                upd2_b[l][None], ln_g[l][None], ln_b[l][None],
                out_ln_g[None], out_ln_b[None], out_W, out_b[None]],
                last=True)

    src = jnp.broadcast_to(
        jnp.arange(BN, dtype=jnp.int32)[:, None], (BN, K)).reshape(-1)
    edge_index = jnp.stack([src, idx_flat.reshape(-1)])
    return out.reshape(B, N, H), edge_index


# speedup vs baseline: 1.2510x; 1.0098x over previous
import functools

import jax
import jax.numpy as jnp
from jax import lax
from jax.experimental import pallas as pl
from jax.experimental.pallas import tpu as pltpu
from jax.experimental.pallas import tpu_sc as plsc

F32 = jnp.float32
N = 2048
H = 64
K = 30
NRBF = 16

RB1 = 256
RBM = 256
RBU = 512

NC, NS = 2, 16
NW = NC * NS
CH = 128

_CSP = 20.0 / 15.0
_INV2W2 = 1.0 / (2.0 * (20.0 / 16.0) ** 2)


def _centers(rows):
    c = lax.broadcasted_iota(jnp.int32, (rows, NRBF), 1)
    return c.astype(F32) * _CSP


def _silu(z):
    return z / (1.0 + jnp.exp(-z))


def _s1_body(c_ref, ct_ref, idx_ref, de_ref, rbf_ref):
    i = pl.program_id(1)
    cr = c_ref[0]
    ct = ct_ref[0]
    sqr = jnp.sum(cr * cr, axis=1, keepdims=True)
    sqc = jnp.sum(ct * ct, axis=0, keepdims=True)
    dot = jnp.dot(cr, ct, preferred_element_type=F32)
    d2 = sqr + sqc - 2.0 * dot
    dist = jnp.sqrt(jnp.maximum(d2, 1e-12))
    col = lax.broadcasted_iota(jnp.int32, (RB1, N), 1)
    rowg = i * RB1 + lax.broadcasted_iota(jnp.int32, (RB1, N), 0)
    D = jnp.where(col == rowg, dist + 1e10, dist)
    mean = jnp.sum(dist, axis=1, keepdims=True) * (1.0 / N)
    rbf_ref[0] = jnp.exp(-(mean - _centers(RB1)) ** 2 * _INV2W2)
    idx_cols, d_cols = [], []
    for _ in range(K):
        mval = jnp.min(D, axis=1, keepdims=True)
        hit = D == mval
        jstar = jnp.min(jnp.where(hit, col, N), axis=1, keepdims=True)
        idx_cols.append(jstar)
        d_cols.append(mval)
        D = jnp.where(col == jstar, 1e30, D)
    idx_ref[0] = jnp.concatenate(idx_cols, axis=1)
    de_ref[0] = jnp.concatenate(d_cols, axis=1)


def _s1_call(coords, coordsT):
    B = coords.shape[0]
    return pl.pallas_call(
        _s1_body,
        grid=(B, N // RB1),
        in_specs=[
            pl.BlockSpec((1, RB1, 3), lambda b, i: (b, i, 0)),
            pl.BlockSpec((1, 3, N), lambda b, i: (b, 0, 0)),
        ],
        out_specs=[
            pl.BlockSpec((1, RB1, K), lambda b, i: (b, i, 0)),
            pl.BlockSpec((1, RB1, K), lambda b, i: (b, i, 0)),
            pl.BlockSpec((1, RB1, NRBF), lambda b, i: (b, i, 0)),
        ],
        out_shape=[
            jax.ShapeDtypeStruct((B, N, K), jnp.int32),
            jax.ShapeDtypeStruct((B, N, K), F32),
            jax.ShapeDtypeStruct((B, N, NRBF), F32),
        ],
    )(coords, coordsT)


def _s2_body(vp_ref, vs_ref, rbf_ref, aw_ref, ab_ref, pwa_ref, pwr_ref,
             pb_ref, w1a_ref, b1_ref, w1b_ref, h_ref, a_ref, bm_ref):
    vp = vp_ref[0]
    vs = vs_ref[0]
    npn = jnp.sqrt(jnp.sum(vp * vp, axis=1, keepdims=True))
    nsn = jnp.sqrt(jnp.sum(vs * vs, axis=1, keepdims=True))
    vpn = vp / jnp.maximum(npn, 1e-12)
    vsn = vs / jnp.maximum(nsn, 1e-12)
    cos = jnp.clip(jnp.sum(vpn * vsn, axis=1, keepdims=True), -1.0, 1.0)
    sin = jnp.sqrt(jnp.maximum(1.0 - cos * cos, 0.0))
    ang = jnp.concatenate([sin, cos, cos, sin], axis=1)
    af = jnp.dot(ang, aw_ref[...], preferred_element_type=F32) + ab_ref[...]
    h = (jnp.dot(af, pwa_ref[...], preferred_element_type=F32)
         + jnp.dot(rbf_ref[0], pwr_ref[...], preferred_element_type=F32)
         + pb_ref[...])
    h_ref[0] = h
    a_ref[0] = jnp.dot(h, w1a_ref[...], preferred_element_type=F32) + b1_ref[...]
    bm_ref[0] = jnp.dot(h, w1b_ref[...], preferred_element_type=F32)


def _s2_call(v1p, v1s, rbfm, aw4, ab, pwa, pwr, pb, w1a, b1, w1b):
    B = v1p.shape[0]
    full = lambda b, i: (0, 0)
    return pl.pallas_call(
        _s2_body,
        grid=(B, N // RBU),
        in_specs=[
            pl.BlockSpec((1, RBU, 3), lambda b, i: (b, i, 0)),
            pl.BlockSpec((1, RBU, 3), lambda b, i: (b, i, 0)),
            pl.BlockSpec((1, RBU, NRBF), lambda b, i: (b, i, 0)),
            pl.BlockSpec((4, H), full),
            pl.BlockSpec((1, H), full),
            pl.BlockSpec((H, H), full),
            pl.BlockSpec((NRBF, H), full),
            pl.BlockSpec((1, H), full),
            pl.BlockSpec((H, H), full),
            pl.BlockSpec((1, H), full),
            pl.BlockSpec((H, H), full),
        ],
        out_specs=[pl.BlockSpec((1, RBU, H), lambda b, i: (b, i, 0))] * 3,
        out_shape=[jax.ShapeDtypeStruct((B, N, H), F32)] * 3,
    )(v1p, v1s, rbfm, aw4, ab, pwa, pwr, pb, w1a, b1, w1b)


def _msg_body(a_ref, bd_ref, de_ref, w1c_ref, w2_ref, b2_ref, m_ref):
    a = a_ref[...]
    de = de_ref[...]
    w1c = w1c_ref[...]
    w2 = w2_ref[...]
    b2 = b2_ref[...]
    cen = _centers(RBM)
    for k in range(K):
        dk = de[:, k:k + 1]
        e16 = jnp.exp(-(dk - cen) ** 2 * _INV2W2)
        z = a + bd_ref[k] + jnp.dot(e16, w1c, preferred_element_type=F32)
        m_ref[k] = jnp.dot(_silu(z), w2, preferred_element_type=F32) + b2


def _msg_call(a, bd, de, w1c, w2, b2):
    BN = a.shape[0]
    full = lambda i: (0, 0)
    return pl.pallas_call(
        _msg_body,
        grid=(BN // RBM,),
        in_specs=[
            pl.BlockSpec((RBM, H), lambda i: (i, 0)),
            pl.BlockSpec((K, RBM, H), lambda i: (0, i, 0)),
            pl.BlockSpec((RBM, K), lambda i: (i, 0)),
            pl.BlockSpec((NRBF, H), full),
            pl.BlockSpec((H, H), full),
            pl.BlockSpec((1, H), full),
        ],
        out_specs=pl.BlockSpec((K, RBM, H), lambda i: (0, i, 0)),
        out_shape=jax.ShapeDtypeStruct((K, BN, H), F32),
    )(a, bd, de, w1c, w2, b2)


def _ln_rows(x, g, b):
    mu = jnp.mean(x, axis=1, keepdims=True)
    xc = x - mu
    var = jnp.mean(xc * xc, axis=1, keepdims=True)
    return xc / jnp.sqrt(var + 1e-5) * g + b


def _upd_common(h_ref, p_ref, u1a_ref, u1b_ref, b1u_ref, u2_ref, b2u_ref,
                g_ref, bb_ref):
    h = h_ref[...]
    aggr = p_ref[0] + p_ref[1]
    z = (jnp.dot(h, u1a_ref[...], preferred_element_type=F32)
         + jnp.dot(aggr, u1b_ref[...], preferred_element_type=F32)
         + b1u_ref[...])
    u = jnp.dot(_silu(z), u2_ref[...], preferred_element_type=F32) + b2u_ref[...]
    return _ln_rows(h + u, g_ref[...], bb_ref[...])


def _upd_mid_body(h_ref, p_ref, u1a_ref, u1b_ref, b1u_ref, u2_ref, b2u_ref,
                  g_ref, bb_ref, w1a_ref, b1n_ref, w1b_ref,
                  hn_ref, an_ref, bmn_ref):
    hn = _upd_common(h_ref, p_ref, u1a_ref, u1b_ref, b1u_ref, u2_ref,
                     b2u_ref, g_ref, bb_ref)
    hn_ref[...] = hn
    an_ref[...] = jnp.dot(hn, w1a_ref[...], preferred_element_type=F32) + b1n_ref[...]
    bmn_ref[...] = jnp.dot(hn, w1b_ref[...], preferred_element_type=F32)


def _upd_last_body(h_ref, p_ref, u1a_ref, u1b_ref, b1u_ref, u2_ref, b2u_ref,
                   g_ref, bb_ref, og_ref, ob_ref, ow_ref, ob2_ref, out_ref):
    hn = _upd_common(h_ref, p_ref, u1a_ref, u1b_ref, b1u_ref, u2_ref,
                     b2u_ref, g_ref, bb_ref)
    y = _ln_rows(hn, og_ref[...], ob_ref[...])
    out_ref[...] = jnp.dot(y, ow_ref[...], preferred_element_type=F32) + ob2_ref[...]


def _upd_call(h, p, consts, last):
    BN = h.shape[0]
    full = lambda i: (0, 0)
    nconst = len(consts)
    in_specs = [
        pl.BlockSpec((RBU, H), lambda i: (i, 0)),
        pl.BlockSpec((2, RBU, H), lambda i: (0, i, 0)),
    ] + [pl.BlockSpec(c.shape, full) for c in consts]
    if last:
        out_specs = pl.BlockSpec((RBU, H), lambda i: (i, 0))
        out_shape = jax.ShapeDtypeStruct((BN, H), F32)
        body = _upd_last_body
    else:
        out_specs = [pl.BlockSpec((RBU, H), lambda i: (i, 0))] * 3
        out_shape = [jax.ShapeDtypeStruct((BN, H), F32)] * 3
        body = _upd_mid_body
    return pl.pallas_call(
        body,
        grid=(BN // RBU,),
        in_specs=in_specs,
        out_specs=out_specs,
        out_shape=out_shape,
    )(h, p, *consts)


def _grp_for(steps):
    for g in (6, 5, 4, 3, 2, 1):
        if steps % (2 * g) == 0:
            return g


def _sc_gather(table, idx3):
    NWk, steps, CHk = idx3.shape
    per = steps * CHk
    E = NWk * per
    GRP = _grp_for(steps)
    NPAIR = steps // (2 * GRP)
    mesh = plsc.VectorSubcoreMesh(core_axis_name="c", subcore_axis_name="s")

    @functools.partial(
        pl.kernel,
        out_type=jax.ShapeDtypeStruct((E, H), F32),
        mesh=mesh,
        compiler_params=pltpu.CompilerParams(use_tc_tiling_on_sc=False),
        scratch_types=[
            pltpu.VMEM((steps, CHk), jnp.int32),
            pltpu.VMEM((GRP * CHk, H), F32),
            pltpu.VMEM((GRP * CHk, H), F32),
            pltpu.SemaphoreType.DMA,
            pltpu.SemaphoreType.DMA,
            pltpu.SemaphoreType.DMA,
            pltpu.SemaphoreType.DMA,
        ],
    )
    def gk(tab_hbm, idx_hbm, out_hbm, idx_all, buf_a, buf_b, gs_a, gs_b,
           os_a, os_b):
        wid = lax.axis_index("s") * NC + lax.axis_index("c")
        base = wid * per
        pltpu.sync_copy(idx_hbm.at[wid], idx_all)

        def issue_group(g, buf, gs):
            for j in range(GRP):
                pltpu.async_copy(tab_hbm.at[idx_all.at[g * GRP + j]],
                                 buf.at[pl.ds(j * CHk, CHk)], gs)

        def drain_group(buf, gs):
            for j in range(GRP):
                pltpu.make_async_copy(tab_hbm.at[idx_all.at[0]],
                                      buf.at[pl.ds(j * CHk, CHk)], gs).wait()

        def out_group(g, buf, osem):
            pltpu.async_copy(
                buf, out_hbm.at[pl.ds(base + g * GRP * CHk, GRP * CHk)], osem)

        def drain_out(buf, osem):
            pltpu.make_async_copy(
                buf, out_hbm.at[pl.ds(base, GRP * CHk)], osem).wait()

        issue_group(0, buf_a, gs_a)
        issue_group(1, buf_b, gs_b)

        def body(p, carry):
            ga = 2 * p
            drain_group(buf_a, gs_a)
            out_group(ga, buf_a, os_a)
            drain_group(buf_b, gs_b)
            out_group(ga + 1, buf_b, os_b)

            @pl.when(p < NPAIR - 1)
            def _():
                drain_out(buf_a, os_a)
                issue_group(ga + 2, buf_a, gs_a)
                drain_out(buf_b, os_b)
                issue_group(ga + 3, buf_b, gs_b)

            return carry

        lax.fori_loop(0, NPAIR, body, 0)
        drain_out(buf_a, os_a)
        drain_out(buf_b, os_b)

    return gk(table, idx3)


def _sc_scatter(m, idx3, zeros):
    NWk, steps, CHk = idx3.shape
    per = steps * CHk
    rps = zeros.shape[0]
    BN = rps * NS
    mesh = plsc.VectorSubcoreMesh(core_axis_name="c", subcore_axis_name="s")

    @functools.partial(
        pl.kernel,
        out_type=jax.ShapeDtypeStruct((NC * BN, H), F32),
        mesh=mesh,
        compiler_params=pltpu.CompilerParams(use_tc_tiling_on_sc=False),
        scratch_types=[
            pltpu.VMEM_SHARED((BN, H), F32),
            pltpu.VMEM((steps, CHk), jnp.int32),
            pltpu.VMEM((CHk, H), F32),
            pltpu.VMEM((CHk, H), F32),
            pltpu.SemaphoreType.DMA,
            pltpu.SemaphoreType.DMA,
        ],
    )
    def sk(m_hbm, idx_hbm, z_hbm, out_hbm, shared, idx_all, m_a, m_b,
           ms_a, ms_b):
        c = lax.axis_index("c")
        s = lax.axis_index("s")
        wid = s * NC + c
        base = wid * per
        pltpu.sync_copy(z_hbm, shared.at[pl.ds(s * rps, rps)])
        pltpu.sync_copy(idx_hbm.at[wid], idx_all)
        plsc.subcore_barrier()

        pltpu.async_copy(m_hbm.at[pl.ds(base, CHk)], m_a, ms_a)
        pltpu.async_copy(m_hbm.at[pl.ds(base + CHk, CHk)], m_b, ms_b)

        def body(q, carry):
            t = 2 * q
            pltpu.make_async_copy(m_hbm.at[pl.ds(base, CHk)], m_a, ms_a).wait()
            pltpu.sync_copy(m_a, shared.at[idx_all.at[t]], add=True)

            @pl.when(q < steps // 2 - 1)
            def _():
                pltpu.async_copy(
                    m_hbm.at[pl.ds(base + (t + 2) * CHk, CHk)], m_a, ms_a)

            pltpu.make_async_copy(m_hbm.at[pl.ds(base, CHk)], m_b, ms_b).wait()
            pltpu.sync_copy(m_b, shared.at[idx_all.at[t + 1]], add=True)

            @pl.when(q < steps // 2 - 1)
            def _():
                pltpu.async_copy(
                    m_hbm.at[pl.ds(base + (t + 3) * CHk, CHk)], m_b, ms_b)

            return carry

        lax.fori_loop(0, steps // 2, body, 0)
        plsc.subcore_barrier()
        pltpu.sync_copy(shared.at[pl.ds(s * rps, rps)],
                        out_hbm.at[pl.ds(c * BN + s * rps, rps)])

    return sk(m, idx3, zeros)


def kernel(coords, mask, angle_W, angle_b, pos_W, pos_b, msg1_W, msg1_b,
           msg2_W, msg2_b, upd1_W, upd1_b, upd2_W, upd2_b, ln_g, ln_b,
           out_ln_g, out_ln_b, out_W, out_b):
    B = coords.shape[0]
    CHB = 120

    coordsT = jnp.transpose(coords, (0, 2, 1))
    v1 = coords[:, 1:] - coords[:, :-1]
    v1p = jnp.pad(v1, ((0, 0), (0, 1), (0, 0)))
    v1s = jnp.pad(v1, ((0, 0), (1, 0), (0, 0)))

    zeros = jnp.zeros((N // NS, H), F32)
    outs, idxs = [], []
    for b in range(B):
        idxg, d_e, rbfm = _s1_call(coords[b:b + 1], coordsT[b:b + 1])
        h, a, bm = _s2_call(v1p[b:b + 1], v1s[b:b + 1], rbfm,
                            angle_W[:4], angle_b[None],
                            pos_W[:H], pos_W[H:], pos_b[None],
                            msg1_W[0, :H], msg1_b[0][None],
                            msg1_W[0, H:2 * H])
        idx_flat = idxg.reshape(N, K)
        dst3 = idx_flat.T.reshape(NW, (N * K) // (NW * CHB), CHB)
        de_flat = d_e.reshape(N, K)
        hf = h.reshape(N, H)
        a = a.reshape(N, H)
        bm = bm.reshape(N, H)
        y_b = None
        for l in range(3):
            bd = _sc_gather(bm, dst3).reshape(K, N, H)
            m = _msg_call(a, bd, de_flat, msg1_W[l, 2 * H:], msg2_W[l],
                          msg2_b[l][None])
            p = _sc_scatter(m.reshape(N * K, H), dst3, zeros).reshape(2, N, H)
            if l < 2:
                hf, a, bm = _upd_call(hf, p, [
                    upd1_W[l, :H], upd1_W[l, H:], upd1_b[l][None], upd2_W[l],
                    upd2_b[l][None], ln_g[l][None], ln_b[l][None],
                    msg1_W[l + 1, :H], msg1_b[l + 1][None],
                    msg1_W[l + 1, H:2 * H]], last=False)
            else:
                y_b = _upd_call(hf, p, [
                    upd1_W[l, :H], upd1_W[l, H:], upd1_b[l][None], upd2_W[l],
                    upd2_b[l][None], ln_g[l][None], ln_b[l][None],
                    out_ln_g[None], out_ln_b[None], out_W, out_b[None]],
                    last=True)
        outs.append(y_b.reshape(1, N, H))
        idxs.append(idx_flat[None] + b * N)

    out = jnp.concatenate(outs, axis=0)
    dst = jnp.stack(idxs).reshape(-1)
    src = jnp.broadcast_to(
        jnp.arange(B * N, dtype=jnp.int32)[:, None], (B * N, K)).reshape(-1)
    edge_index = jnp.stack([src, dst])
    return out, edge_index
